# baseline (speedup 1.0000x reference)
import functools

import jax
import jax.numpy as jnp
from jax.experimental import pallas as pl
from jax.experimental.pallas import tpu as pltpu

_BM = 1024
_KSPLIT = 4


def _hollow_matmul_kernel(*refs, bm, kq, ksplit):
    w_refs = refs[:ksplit]
    x_refs = refs[ksplit:2 * ksplit]
    o_ref = refs[2 * ksplit]
    i = pl.program_id(0)
    acc = None
    for q in range(ksplit):
        w = w_refs[q][...]
        r = jax.lax.broadcasted_iota(jnp.int32, w.shape, 0)
        c = jax.lax.broadcasted_iota(jnp.int32, w.shape, 1)
        w = jnp.where(c == r + (i * bm - q * kq), jnp.zeros_like(w), w)
        d = jnp.dot(w, x_refs[q][...], preferred_element_type=jnp.float32)
        acc = d if acc is None else acc + d
    o_ref[...] = acc


def kernel(weight, x):
    n, n2 = weight.shape
    assert n == n2
    d = x.shape[1]
    bm = _BM
    ksplit = _KSPLIT
    kq = n // ksplit
    assert n % bm == 0 and n % ksplit == 0

    def _w_spec(q):
        return pl.BlockSpec((bm, kq), lambda i, q=q: (i, q))

    def _x_spec(q):
        return pl.BlockSpec((kq, d), lambda i, q=q: (q, 0))

    out = pl.pallas_call(
        functools.partial(_hollow_matmul_kernel, bm=bm, kq=kq, ksplit=ksplit),
        grid=(n // bm,),
        in_specs=[_w_spec(q) for q in range(ksplit)]
        + [_x_spec(q) for q in range(ksplit)],
        out_specs=pl.BlockSpec((bm, d), lambda i: (i, 0)),
        out_shape=jax.ShapeDtypeStruct((n, d), jnp.float32),
        compiler_params=pltpu.CompilerParams(
            dimension_semantics=("parallel",),
            vmem_limit_bytes=64 * 1024 * 1024,
        ),
    )(*([weight] * ksplit + [x] * ksplit))
    return weight, out


# speedup vs baseline: 1.0135x; 1.0135x over previous
import functools

import jax
import jax.numpy as jnp
from jax.experimental import pallas as pl
from jax.experimental.pallas import tpu as pltpu

_BM = 1024


def _hollow_matmul_kernel(w_ref, x_ref, o_ref, *, bm):
    i = pl.program_id(0)
    w = w_ref[...]
    r = jax.lax.broadcasted_iota(jnp.int32, w.shape, 0)
    c = jax.lax.broadcasted_iota(jnp.int32, w.shape, 1)
    w = jnp.where(c == r + i * bm, jnp.zeros_like(w), w)
    o_ref[...] = jnp.dot(w, x_ref[...], preferred_element_type=jnp.float32)


def kernel(weight, x):
    n, n2 = weight.shape
    assert n == n2
    d = x.shape[1]
    bm = _BM
    assert n % bm == 0

    out = pl.pallas_call(
        functools.partial(_hollow_matmul_kernel, bm=bm),
        grid=(n // bm,),
        in_specs=[
            pl.BlockSpec((bm, n), lambda i: (i, 0)),
            pl.BlockSpec((n, d), lambda i: (0, 0)),
        ],
        out_specs=pl.BlockSpec((bm, d), lambda i: (i, 0)),
        out_shape=jax.ShapeDtypeStruct((n, d), jnp.float32),
        compiler_params=pltpu.CompilerParams(
            dimension_semantics=("parallel",),
            vmem_limit_bytes=64 * 1024 * 1024,
        ),
    )(weight, x)
    return weight, out
